# Initial kernel scaffold; baseline (speedup 1.0000x reference)
import jax
import jax.numpy as jnp
from jax.experimental import pallas as pl


def kernel(adj, x, W, b, gamma, beta):
    raise NotImplementedError("write your pallas kernel here")


# speedup vs baseline: 12.7287x; 12.7287x over previous
import functools

import jax
import jax.numpy as jnp
from jax import lax
from jax.experimental import pallas as pl
from jax.experimental.pallas import tpu as pltpu
from jax.experimental.pallas import tpu_sc as plsc

N = 10000
E = 320000
D = 128

NC = 2
NS = 16
NW = NC * NS

CP = 128
K = (E // NW + CP - 1) // CP
EP = NW * K * CP
NPAD = 10112
RPT = NPAD // NS
DEGW = 16

_mesh = plsc.VectorSubcoreMesh(core_axis_name="c", subcore_axis_name="s")


def _deg_body(src3, dst3, ones_hbm, zeros_hbm, out_hbm,
              sidx, didx, ones_v, acc, sem):
    cid = lax.axis_index("c")
    sid = lax.axis_index("s")
    wid = cid * NS + sid
    r0 = sid * RPT
    pltpu.sync_copy(zeros_hbm.at[pl.ds(r0, RPT)], acc.at[pl.ds(r0, RPT)])
    pltpu.sync_copy(src3.at[wid], sidx)
    pltpu.sync_copy(dst3.at[wid], didx)
    pltpu.sync_copy(ones_hbm, ones_v)
    plsc.subcore_barrier()

    def body_s(j, _):
        pltpu.sync_copy(ones_v, acc.at[sidx.at[j]], add=True)
        return _

    lax.fori_loop(0, K, body_s, None)
    plsc.subcore_barrier()
    pltpu.sync_copy(acc.at[pl.ds(r0, RPT)], out_hbm.at[cid, 0, pl.ds(r0, RPT)])
    pltpu.sync_copy(zeros_hbm.at[pl.ds(r0, RPT)], acc.at[pl.ds(r0, RPT)])
    plsc.subcore_barrier()

    def body_d(j, _):
        pltpu.sync_copy(ones_v, acc.at[didx.at[j]], add=True)
        return _

    lax.fori_loop(0, K, body_d, None)
    plsc.subcore_barrier()
    pltpu.sync_copy(acc.at[pl.ds(r0, RPT)], out_hbm.at[cid, 1, pl.ds(r0, RPT)])


_deg_kernel = pl.kernel(
    _deg_body,
    out_type=jax.ShapeDtypeStruct((NC, 2, NPAD, D), jnp.float32),
    mesh=_mesh,
    scratch_types=[
        pltpu.VMEM((K, CP), jnp.int32),
        pltpu.VMEM((K, CP), jnp.int32),
        pltpu.VMEM((CP, D), jnp.float32),
        pltpu.VMEM_SHARED((NPAD, D), jnp.float32),
        pltpu.SemaphoreType.DMA,
    ],
)


def _agg_body(h_hbm, src3, dst3, zeros_hbm, out_hbm,
              sidx, didx, rows, acc, sem):
    cid = lax.axis_index("c")
    sid = lax.axis_index("s")
    wid = cid * NS + sid
    r0 = sid * RPT
    pltpu.sync_copy(zeros_hbm.at[pl.ds(r0, RPT)], acc.at[pl.ds(r0, RPT)])
    pltpu.sync_copy(src3.at[wid], sidx)
    pltpu.sync_copy(dst3.at[wid], didx)
    plsc.subcore_barrier()

    def body(j, _):
        pltpu.async_copy(h_hbm.at[sidx.at[j]], rows, sem).wait()
        pltpu.sync_copy(rows, acc.at[didx.at[j]], add=True)
        return _

    lax.fori_loop(0, K, body, None)
    plsc.subcore_barrier()
    pltpu.sync_copy(acc.at[pl.ds(r0, RPT)], out_hbm.at[cid, pl.ds(r0, RPT)])


_agg_kernel = pl.kernel(
    _agg_body,
    out_type=jax.ShapeDtypeStruct((NC, NPAD, D), jnp.float32),
    mesh=_mesh,
    scratch_types=[
        pltpu.VMEM((K, CP), jnp.int32),
        pltpu.VMEM((K, CP), jnp.int32),
        pltpu.VMEM((CP, D), jnp.float32),
        pltpu.VMEM_SHARED((NPAD, D), jnp.float32),
        pltpu.SemaphoreType.DMA,
    ],
)


def _h_body(x_ref, w_ref, d0_ref, d1_ref, o_ref):
    deg = d0_ref[...] + d1_ref[...]
    ns = lax.rsqrt(jnp.maximum(deg, 1.0))
    o_ref[...] = jnp.dot(x_ref[...] * ns, w_ref[...],
                         preferred_element_type=jnp.float32)


_NB = 10
_BR = N // _NB


def _h_kernel(x, W, d0, d1):
    return pl.pallas_call(
        _h_body,
        out_shape=jax.ShapeDtypeStruct((N, D), jnp.float32),
        grid=(_NB,),
        in_specs=[
            pl.BlockSpec((_BR, D), lambda i: (i, 0)),
            pl.BlockSpec((D, D), lambda i: (0, 0)),
            pl.BlockSpec((_BR, 1), lambda i: (i, 0)),
            pl.BlockSpec((_BR, 1), lambda i: (i, 0)),
        ],
        out_specs=pl.BlockSpec((_BR, D), lambda i: (i, 0)),
    )(x, W, d0, d1)


def _ln_body(s0_ref, s1_ref, d0_ref, d1_ref, b_ref, g_ref, be_ref, o_ref):
    deg = d0_ref[...] + d1_ref[...]
    nd = lax.rsqrt(jnp.maximum(deg, 1.0))
    agg = (s0_ref[...] + s1_ref[...]) * nd + b_ref[...]
    mean = jnp.mean(agg, axis=-1, keepdims=True)
    cen = agg - mean
    var = jnp.mean(cen * cen, axis=-1, keepdims=True)
    normed = cen * lax.rsqrt(var + 1e-5) * g_ref[...] + be_ref[...]
    o_ref[...] = jnp.maximum(normed, 0.0)


def _ln_kernel(s0, s1, d0, d1, b, gamma, beta):
    return pl.pallas_call(
        _ln_body,
        out_shape=jax.ShapeDtypeStruct((N, D), jnp.float32),
        grid=(_NB,),
        in_specs=[
            pl.BlockSpec((_BR, D), lambda i: (i, 0)),
            pl.BlockSpec((_BR, D), lambda i: (i, 0)),
            pl.BlockSpec((_BR, 1), lambda i: (i, 0)),
            pl.BlockSpec((_BR, 1), lambda i: (i, 0)),
            pl.BlockSpec((1, D), lambda i: (0, 0)),
            pl.BlockSpec((1, D), lambda i: (0, 0)),
            pl.BlockSpec((1, D), lambda i: (0, 0)),
        ],
        out_specs=pl.BlockSpec((_BR, D), lambda i: (i, 0)),
    )(s0, s1, d0, d1, b, gamma, beta)


@jax.jit
def kernel(adj, x, W, b, gamma, beta):
    src = adj[:, 0]
    dst = adj[:, 1]
    pad = EP - E
    trash = N + (jnp.arange(pad, dtype=jnp.int32) % DEGW)
    zeros_e = jnp.zeros((pad,), dtype=jnp.int32)
    src3_t = jnp.concatenate([src, trash]).reshape(NW, K, CP)
    src3_z = jnp.concatenate([src, zeros_e]).reshape(NW, K, CP)
    dst3_t = jnp.concatenate([dst, trash]).reshape(NW, K, CP)

    ones_w = jnp.ones((CP, D), dtype=jnp.float32)
    zeros_d = jnp.zeros((NPAD, D), dtype=jnp.float32)

    dd = _deg_kernel(src3_t, dst3_t, ones_w, zeros_d)
    deg_out0 = dd[0, 0, :N, 0:1]
    deg_out1 = dd[1, 0, :N, 0:1]
    deg_in0 = dd[0, 1, :N, 0:1]
    deg_in1 = dd[1, 1, :N, 0:1]

    h = _h_kernel(x, W, deg_out0, deg_out1)

    part = _agg_kernel(h, src3_z, dst3_t, zeros_d)
    s0 = part[0, :N]
    s1 = part[1, :N]

    return _ln_kernel(s0, s1, deg_in0, deg_in1,
                      b.reshape(1, D), gamma.reshape(1, D),
                      beta.reshape(1, D))


# speedup vs baseline: 24.5206x; 1.9264x over previous
import jax
import jax.numpy as jnp
from jax import lax
from jax.experimental import pallas as pl
from jax.experimental.pallas import tpu as pltpu
from jax.experimental.pallas import tpu_sc as plsc

N = 10000
E = 320000
D = 128

NC = 2
NS = 16
NW = NC * NS

NPAD = 10112
RPT = NPAD // NS
DEGW = 16

CP = 128
K = (E // NW + CP - 1) // CP
EP = NW * K * CP

CA = 64
KA = (E // NW + CA - 1) // CA
EA = NW * KA * CA

_mesh = plsc.VectorSubcoreMesh(core_axis_name="c", subcore_axis_name="s")
_no_tiling = pltpu.CompilerParams(use_tc_tiling_on_sc=False)


def _deg_body(src3, ones_hbm, zeros_hbm, out_hbm, sidx, ones_v, cnt, sem):
    cid = lax.axis_index("c")
    sid = lax.axis_index("s")
    wid = cid * NS + sid
    r0 = sid * RPT
    pltpu.sync_copy(zeros_hbm.at[pl.ds(r0, RPT)], cnt.at[pl.ds(r0, RPT)])
    pltpu.sync_copy(src3.at[wid], sidx)
    pltpu.sync_copy(ones_hbm, ones_v)
    plsc.subcore_barrier()

    def body(j, carry):
        pltpu.sync_copy(ones_v, cnt.at[sidx.at[j]], add=True)
        return carry

    lax.fori_loop(0, K, body, None)
    plsc.subcore_barrier()
    pltpu.sync_copy(cnt.at[pl.ds(r0, RPT)], out_hbm.at[cid, pl.ds(r0, RPT)])


_deg_kernel = pl.kernel(
    _deg_body,
    out_type=jax.ShapeDtypeStruct((NC, NPAD, DEGW), jnp.float32),
    mesh=_mesh,
    compiler_params=_no_tiling,
    scratch_types=[
        pltpu.VMEM((K, CP), jnp.int32),
        pltpu.VMEM((CP, DEGW), jnp.float32),
        pltpu.VMEM_SHARED((NPAD, DEGW), jnp.float32),
        pltpu.SemaphoreType.DMA,
    ],
)


def _agg_body(h_hbm, src3, dst3, ones_hbm, zeros_hbm, zeros16_hbm,
              out_hbm, cnt_hbm,
              sidx, didx, rows0, rows1, ones_v, acc, cnt,
              gs0, gs1, ss0, ss1, cs):
    cid = lax.axis_index("c")
    sid = lax.axis_index("s")
    wid = cid * NS + sid
    r0 = sid * RPT
    pltpu.sync_copy(zeros_hbm.at[pl.ds(r0, RPT)], acc.at[pl.ds(r0, RPT)])
    pltpu.sync_copy(zeros16_hbm.at[pl.ds(r0, RPT)], cnt.at[pl.ds(r0, RPT)])
    pltpu.sync_copy(src3.at[wid], sidx)
    pltpu.sync_copy(dst3.at[wid], didx)
    pltpu.sync_copy(ones_hbm, ones_v)
    plsc.subcore_barrier()

    pltpu.async_copy(h_hbm.at[sidx.at[0]], rows0, gs0)

    def body(t, carry):
        j0 = 2 * t
        pltpu.async_copy(h_hbm.at[sidx.at[j0 + 1]], rows1, gs1)
        pltpu.async_copy(ones_v, cnt.at[didx.at[j0]], cs, add=True)
        pltpu.make_async_copy(h_hbm.at[sidx.at[j0]], rows0, gs0).wait()
        pltpu.async_copy(rows0, acc.at[didx.at[j0]], ss0, add=True)
        pltpu.make_async_copy(ones_v, cnt.at[didx.at[j0]], cs).wait()
        pltpu.async_copy(ones_v, cnt.at[didx.at[j0 + 1]], cs, add=True)
        pltpu.make_async_copy(rows0, acc.at[didx.at[j0]], ss0).wait()
        pltpu.async_copy(h_hbm.at[sidx.at[j0 + 2]], rows0, gs0)
        pltpu.make_async_copy(h_hbm.at[sidx.at[j0 + 1]], rows1, gs1).wait()
        pltpu.async_copy(rows1, acc.at[didx.at[j0 + 1]], ss1, add=True)
        pltpu.make_async_copy(ones_v, cnt.at[didx.at[j0 + 1]], cs).wait()
        pltpu.make_async_copy(rows1, acc.at[didx.at[j0 + 1]], ss1).wait()
        return carry

    lax.fori_loop(0, (KA - 1) // 2, body, None)
    jlast = KA - 1
    pltpu.make_async_copy(h_hbm.at[sidx.at[jlast]], rows0, gs0).wait()
    pltpu.sync_copy(rows0, acc.at[didx.at[jlast]], add=True)
    pltpu.sync_copy(ones_v, cnt.at[didx.at[jlast]], add=True)
    plsc.subcore_barrier()
    pltpu.sync_copy(acc.at[pl.ds(r0, RPT)], out_hbm.at[cid, pl.ds(r0, RPT)])
    pltpu.sync_copy(cnt.at[pl.ds(r0, RPT)], cnt_hbm.at[cid, pl.ds(r0, RPT)])


_agg_kernel = pl.kernel(
    _agg_body,
    out_type=(
        jax.ShapeDtypeStruct((NC, NPAD, D), jnp.float32),
        jax.ShapeDtypeStruct((NC, NPAD, DEGW), jnp.float32),
    ),
    mesh=_mesh,
    compiler_params=_no_tiling,
    scratch_types=[
        pltpu.VMEM((KA, CA), jnp.int32),
        pltpu.VMEM((KA, CA), jnp.int32),
        pltpu.VMEM((CA, D), jnp.float32),
        pltpu.VMEM((CA, D), jnp.float32),
        pltpu.VMEM((CA, DEGW), jnp.float32),
        pltpu.VMEM_SHARED((NPAD, D), jnp.float32),
        pltpu.VMEM_SHARED((NPAD, DEGW), jnp.float32),
        pltpu.SemaphoreType.DMA,
        pltpu.SemaphoreType.DMA,
        pltpu.SemaphoreType.DMA,
        pltpu.SemaphoreType.DMA,
        pltpu.SemaphoreType.DMA,
    ],
)


def _h_body(x_ref, w_ref, d0_ref, d1_ref, o_ref):
    deg = d0_ref[0, :, 0:1] + d1_ref[0, :, 0:1]
    ns = lax.rsqrt(jnp.maximum(deg, 1.0))
    o_ref[...] = jnp.dot(x_ref[...] * ns, w_ref[...],
                         preferred_element_type=jnp.float32)


_NB = 10
_BR = N // _NB


def _h_kernel(x, W, dd):
    return pl.pallas_call(
        _h_body,
        out_shape=jax.ShapeDtypeStruct((N, D), jnp.float32),
        grid=(_NB,),
        in_specs=[
            pl.BlockSpec((_BR, D), lambda i: (i, 0)),
            pl.BlockSpec((D, D), lambda i: (0, 0)),
            pl.BlockSpec((1, _BR, DEGW), lambda i: (0, i, 0)),
            pl.BlockSpec((1, _BR, DEGW), lambda i: (1, i, 0)),
        ],
        out_specs=pl.BlockSpec((_BR, D), lambda i: (i, 0)),
    )(x, W, dd, dd)


def _ln_body(s0_ref, s1_ref, d0_ref, d1_ref, b_ref, g_ref, be_ref, o_ref):
    deg = d0_ref[0, :, 0:1] + d1_ref[0, :, 0:1]
    nd = lax.rsqrt(jnp.maximum(deg, 1.0))
    agg = (s0_ref[0] + s1_ref[0]) * nd + b_ref[...]
    mean = jnp.mean(agg, axis=-1, keepdims=True)
    cen = agg - mean
    var = jnp.mean(cen * cen, axis=-1, keepdims=True)
    normed = cen * lax.rsqrt(var + 1e-5) * g_ref[...] + be_ref[...]
    o_ref[...] = jnp.maximum(normed, 0.0)


def _ln_kernel(part, cc, b, gamma, beta):
    return pl.pallas_call(
        _ln_body,
        out_shape=jax.ShapeDtypeStruct((N, D), jnp.float32),
        grid=(_NB,),
        in_specs=[
            pl.BlockSpec((1, _BR, D), lambda i: (0, i, 0)),
            pl.BlockSpec((1, _BR, D), lambda i: (1, i, 0)),
            pl.BlockSpec((1, _BR, DEGW), lambda i: (0, i, 0)),
            pl.BlockSpec((1, _BR, DEGW), lambda i: (1, i, 0)),
            pl.BlockSpec((1, D), lambda i: (0, 0)),
            pl.BlockSpec((1, D), lambda i: (0, 0)),
            pl.BlockSpec((1, D), lambda i: (0, 0)),
        ],
        out_specs=pl.BlockSpec((_BR, D), lambda i: (i, 0)),
    )(part, part, cc, cc, b, gamma, beta)


@jax.jit
def kernel(adj, x, W, b, gamma, beta):
    src = adj[:, 0]
    dst = adj[:, 1]
    padk = EP - E
    trash_k = N + (jnp.arange(padk, dtype=jnp.int32) % DEGW)
    pada = EA - E
    trash_a = N + (jnp.arange(pada, dtype=jnp.int32) % DEGW)
    zeros_a = jnp.zeros((pada,), dtype=jnp.int32)
    src3_deg = jnp.concatenate([src, trash_k]).reshape(NW, K, CP)
    src3_agg = jnp.concatenate([src, zeros_a]).reshape(NW, KA, CA)
    dst3_agg = jnp.concatenate([dst, trash_a]).reshape(NW, KA, CA)

    ones_k = jnp.ones((CP, DEGW), dtype=jnp.float32)
    ones_a = jnp.ones((CA, DEGW), dtype=jnp.float32)
    zeros16 = jnp.zeros((NPAD, DEGW), dtype=jnp.float32)
    zeros_d = jnp.zeros((NPAD, D), dtype=jnp.float32)

    dd = _deg_kernel(src3_deg, ones_k, zeros16)

    h = _h_kernel(x, W, dd)

    part, cc = _agg_kernel(h, src3_agg, dst3_agg, ones_a, zeros_d, zeros16)

    return _ln_kernel(part, cc, b.reshape(1, D), gamma.reshape(1, D),
                      beta.reshape(1, D))
